# Initial kernel scaffold; baseline (speedup 1.0000x reference)
import jax
import jax.numpy as jnp
from jax.experimental import pallas as pl


def kernel(x, edge_index, batch, Wl1, bl1, Wr1, Wl2, bl2, Wr2, W1, b1, W2, b2, W3, b3):
    raise NotImplementedError("write your pallas kernel here")


# speedup vs baseline: 3.1608x; 3.1608x over previous
import functools

import jax
import jax.numpy as jnp
from jax import lax
from jax.experimental import pallas as pl
from jax.experimental.pallas import tpu as pltpu
from jax.experimental.pallas import tpu_sc as plsc

_N = 10000
_D = 128
_E = 320000
_G = 64

_NC = 2
_NS = 16
_NW = _NC * _NS

_CHUNK = 128
_NCH = 80
_HCH = _NCH // 2
_EPAD = _NW * _NCH * _CHUNK
_RPAD = 10240
_RPT = _RPAD // _NS
_BLK = 512


@functools.lru_cache(maxsize=None)
def _edge_agg_kernel():
    return functools.partial(
        pl.kernel,
        out_type=jax.ShapeDtypeStruct((_NC, _RPAD, _D), jnp.float32),
        mesh=plsc.VectorSubcoreMesh(
            core_axis_name="c", subcore_axis_name="s", num_cores=_NC, num_subcores=_NS
        ),
        scratch_types=[
            pltpu.VMEM((_HCH, _CHUNK), jnp.int32),
            pltpu.VMEM((_HCH, _CHUNK), jnp.int32),
            pltpu.VMEM((_CHUNK, _D), jnp.float32),
            pltpu.VMEM((_CHUNK, _D), jnp.float32),
            pltpu.VMEM_SHARED((_RPAD, _D), jnp.float32),
            pltpu.SemaphoreType.DMA,
            pltpu.SemaphoreType.DMA,
        ],
    )(_edge_agg_body)


def _edge_agg_body(feat, srcs, dsts, zeros, out, src_v, dst_v, rows_a, rows_b, acc, sem_a, sem_b):
    cid = lax.axis_index("c")
    sid = lax.axis_index("s")
    wid = sid * _NC + cid
    base = sid * _RPT

    pltpu.sync_copy(zeros.at[pl.ds(base, _RPT)], acc.at[pl.ds(base, _RPT)])
    plsc.subcore_barrier()

    for h in range(_NCH // _HCH):
        pltpu.sync_copy(srcs.at[wid, pl.ds(h * _HCH, _HCH)], src_v)
        pltpu.sync_copy(dsts.at[wid, pl.ds(h * _HCH, _HCH)], dst_v)
        pltpu.async_copy(feat.at[src_v.at[0]], rows_a, sem_a)

        def chunk(j, carry):
            @pl.when(lax.rem(j, 2) == 0)
            def _():
                @pl.when(j + 1 < _HCH)
                def _():
                    pltpu.async_copy(feat.at[src_v.at[j + 1]], rows_b, sem_b)
                pltpu.make_async_copy(feat.at[src_v.at[j]], rows_a, sem_a).wait()
                pltpu.sync_copy(rows_a, acc.at[dst_v.at[j]], add=True)

            @pl.when(lax.rem(j, 2) == 1)
            def _():
                @pl.when(j + 1 < _HCH)
                def _():
                    pltpu.async_copy(feat.at[src_v.at[j + 1]], rows_a, sem_a)
                pltpu.make_async_copy(feat.at[src_v.at[j]], rows_b, sem_b).wait()
                pltpu.sync_copy(rows_b, acc.at[dst_v.at[j]], add=True)

            return carry

        lax.fori_loop(0, _HCH, chunk, 0)
    plsc.subcore_barrier()
    pltpu.sync_copy(acc.at[pl.ds(base, _RPT)], out.at[cid, pl.ds(base, _RPT)])


def _dot_t(a, w):
    return lax.dot_general(a, w, (((1,), (1,)), ((), ())),
                           preferred_element_type=jnp.float32)


def _conv_body(agg_ref, feat_ref, wl_ref, bl_ref, wr_ref, out_ref):
    a = agg_ref[0] + agg_ref[1]
    h = _dot_t(a, wl_ref[...]) + bl_ref[...] + _dot_t(feat_ref[...], wr_ref[...])
    out_ref[...] = jnp.maximum(h, 0.0)


_conv_tc = pl.pallas_call(
    _conv_body,
    grid=(_RPAD // _BLK,),
    in_specs=[
        pl.BlockSpec((_NC, _BLK, _D), lambda i: (0, i, 0)),
        pl.BlockSpec((_BLK, _D), lambda i: (i, 0)),
        pl.BlockSpec((_D, _D), lambda i: (0, 0)),
        pl.BlockSpec((1, _D), lambda i: (0, 0)),
        pl.BlockSpec((_D, _D), lambda i: (0, 0)),
    ],
    out_specs=pl.BlockSpec((_BLK, _D), lambda i: (i, 0)),
    out_shape=jax.ShapeDtypeStruct((_RPAD, _D), jnp.float32),
)


def _conv_pool_body(agg_ref, feat_ref, batch_ref, wl_ref, bl_ref, wr_ref,
                    w1_ref, b1_ref, w2_ref, b2_ref, w3_ref, b3_ref,
                    out_ref, pooled):
    i = pl.program_id(0)
    a = agg_ref[0] + agg_ref[1]
    h = jnp.maximum(
        _dot_t(a, wl_ref[...]) + bl_ref[...] + _dot_t(feat_ref[...], wr_ref[...]),
        0.0,
    )
    bb = batch_ref[0, 0, :]
    onehot = (bb[None, :] == lax.broadcasted_iota(jnp.int32, (_G, _BLK), 0)
              ).astype(jnp.float32)
    contrib = jnp.dot(onehot, h, preferred_element_type=jnp.float32)

    @pl.when(i == 0)
    def _():
        pooled[...] = contrib

    @pl.when(i > 0)
    def _():
        pooled[...] = pooled[...] + contrib

    @pl.when(i == pl.num_programs(0) - 1)
    def _():
        z = jnp.maximum(_dot_t(pooled[...], w1_ref[...]) + b1_ref[...], 0.0)
        z = jnp.maximum(_dot_t(z, w2_ref[...]) + b2_ref[...], 0.0)
        out_ref[...] = _dot_t(z, w3_ref[...]) + b3_ref[...]


_conv_pool_tc = pl.pallas_call(
    _conv_pool_body,
    grid=(_RPAD // _BLK,),
    in_specs=[
        pl.BlockSpec((_NC, _BLK, _D), lambda i: (0, i, 0)),
        pl.BlockSpec((_BLK, _D), lambda i: (i, 0)),
        pl.BlockSpec((1, 1, _BLK), lambda i: (i, 0, 0)),
        pl.BlockSpec((_D, _D), lambda i: (0, 0)),
        pl.BlockSpec((1, _D), lambda i: (0, 0)),
        pl.BlockSpec((_D, _D), lambda i: (0, 0)),
        pl.BlockSpec((64, _D), lambda i: (0, 0)),
        pl.BlockSpec((1, 64), lambda i: (0, 0)),
        pl.BlockSpec((32, 64), lambda i: (0, 0)),
        pl.BlockSpec((1, 32), lambda i: (0, 0)),
        pl.BlockSpec((10, 32), lambda i: (0, 0)),
        pl.BlockSpec((1, 10), lambda i: (0, 0)),
    ],
    out_specs=pl.BlockSpec((_G, 10), lambda i: (0, 0)),
    out_shape=jax.ShapeDtypeStruct((_G, 10), jnp.float32),
    scratch_shapes=[pltpu.VMEM((_G, _D), jnp.float32)],
)


def kernel(x, edge_index, batch, Wl1, bl1, Wr1, Wl2, bl2, Wr2, W1, b1, W2, b2, W3, b3):
    src = edge_index[0]
    dst = edge_index[1]
    pad_e = _EPAD - _E
    srcs = jnp.concatenate([src, jnp.zeros((pad_e,), jnp.int32)]).reshape(_NW, _NCH, _CHUNK)
    dsts = jnp.concatenate([dst, jnp.full((pad_e,), _N, jnp.int32)]).reshape(_NW, _NCH, _CHUNK)
    xp = jnp.pad(x, ((0, _RPAD - _N), (0, 0)))
    batch_r = jnp.pad(batch, (0, _RPAD - _N), constant_values=_G).reshape(
        _RPAD // _BLK, 1, _BLK)
    zeros = jnp.zeros((_RPAD, _D), jnp.float32)

    edge_agg = _edge_agg_kernel()
    agg1 = edge_agg(xp, srcs, dsts, zeros)
    h1 = _conv_tc(agg1, xp, Wl1, bl1.reshape(1, -1), Wr1)
    agg2 = edge_agg(h1, srcs, dsts, zeros)
    out = _conv_pool_tc(agg2, h1, batch_r, Wl2, bl2.reshape(1, -1), Wr2,
                        W1, b1.reshape(1, -1), W2, b2.reshape(1, -1),
                        W3, b3.reshape(1, -1))
    return out
